# baseline (speedup 1.0000x reference)
import jax
import jax.numpy as jnp
from jax.experimental import pallas as pl
from jax.experimental.pallas import tpu as pltpu

N_LINKS = 10000
N_PATHS = 50000
E = 800000
L = 32
P = 32
RU = 256
T = 4
MAXSEQ = 16


def _gru(x, h, Wx, Wh, bx, bh):
    H = h.shape[-1]
    xs = x @ Wx + bx
    hs = h @ Wh + bh
    z = jax.nn.sigmoid(xs[..., :H] + hs[..., :H])
    r = jax.nn.sigmoid(xs[..., H:2 * H] + hs[..., H:2 * H])
    hh = jnp.tanh(xs[..., 2 * H:] + r * hs[..., 2 * H:])
    return z * h + (1.0 - z) * hh


_SELU_SCALE = 1.0507009873554805
_SELU_ALPHA = 1.6732632423543772


def _selu(x):
    return _SELU_SCALE * jnp.where(x > 0, x, _SELU_ALPHA * (jnp.exp(x) - 1.0))


def _mlp_kernel(x_ref, w1_ref, b1_ref, w2_ref, b2_ref, w3_ref, b3_ref, o_ref):
    x = x_ref[...]
    r = x @ w1_ref[...] + b1_ref[...]
    r = _selu(r)
    r = r @ w2_ref[...] + b2_ref[...]
    r = jax.nn.relu(r)
    o_ref[...] = r @ w3_ref[...] + b3_ref[...]


def _readout(path_state, W1, b1, W2, b2, W3, b3):
    BLK = 2000
    grid = (N_PATHS // BLK,)
    return pl.pallas_call(
        _mlp_kernel,
        grid=grid,
        in_specs=[
            pl.BlockSpec((BLK, P), lambda i: (i, 0)),
            pl.BlockSpec((P, RU), lambda i: (0, 0)),
            pl.BlockSpec((RU,), lambda i: (0,)),
            pl.BlockSpec((RU, RU), lambda i: (0, 0)),
            pl.BlockSpec((RU,), lambda i: (0,)),
            pl.BlockSpec((RU, 1), lambda i: (0, 0)),
            pl.BlockSpec((1,), lambda i: (0,)),
        ],
        out_specs=pl.BlockSpec((BLK, 1), lambda i: (i, 0)),
        out_shape=jax.ShapeDtypeStruct((N_PATHS, 1), jnp.float32),
    )(path_state, W1, b1, W2, b2, W3, b3)


def kernel(links, paths, sequences, link_capacity, bandwith,
           path_Wx, path_Wh, path_bx, path_bh,
           link_Wx, link_Wh, link_bx, link_bh,
           W1, b1, W2, b2, W3, b3):
    max_len = jnp.max(sequences) + 1
    lens = jax.ops.segment_sum(jnp.ones_like(paths), paths, num_segments=N_PATHS)
    seq_idx = jnp.arange(MAXSEQ)
    mask = (seq_idx[None, :] < lens[:, None]) & (seq_idx[None, :] < max_len)
    link_state = jnp.concatenate(
        [link_capacity[:, None], jnp.zeros((N_LINKS, L - 1), jnp.float32)], axis=1)
    path_state = jnp.concatenate(
        [bandwith[:, None], jnp.zeros((N_PATHS, P - 1), jnp.float32)], axis=1)

    def step(h, xm):
        x_t, m_t = xm
        h_new = _gru(x_t, h, path_Wx, path_Wh, path_bx, path_bh)
        h_out = jnp.where(m_t[:, None], h_new, h)
        return h_out, h_out

    for _ in range(T):
        h_tild = link_state[links]
        link_inputs = jnp.zeros((N_PATHS, MAXSEQ, L), jnp.float32)
        link_inputs = link_inputs.at[paths, sequences].add(h_tild)
        xs = jnp.swapaxes(link_inputs, 0, 1)
        ms = jnp.swapaxes(mask, 0, 1)
        path_state, outs = jax.lax.scan(step, path_state, (xs, ms))
        outs = jnp.swapaxes(outs, 0, 1)
        m = outs[paths, sequences]
        agg = jax.ops.segment_sum(m, links, num_segments=N_LINKS)
        link_state = _gru(agg, link_state, link_Wx, link_Wh, link_bx, link_bh)

    return _readout(path_state, W1, b1, W2, b2, W3, b3)


# speedup vs baseline: 3.6533x; 3.6533x over previous
import functools

import jax
import jax.numpy as jnp
from jax import lax
from jax.experimental import pallas as pl
from jax.experimental.pallas import tpu as pltpu
from jax.experimental.pallas import tpu_sc as plsc

NL = 10000
NP = 50000
E = 800000
L = 32
P = 32
RU = 256
T = 4
MS = 16

NLP = 10112
EPAD = 819200
STRIPS = 200
EPW = 25600
CP = 1000
NCH = 50
NCH0 = 25
CROWS = MS * CP
TRASH = CROWS
HCH = 25600
HROWS = 25728
HTRASH = HCH
NPH = 2 * HCH
AR = NLP // 16

_SC_PARAMS = pltpu.CompilerParams(use_tc_tiling_on_sc=False)


@functools.lru_cache(maxsize=1)
def _get_mesh():
    return plsc.VectorSubcoreMesh(core_axis_name="c", subcore_axis_name="s",
                                  num_cores=2, num_subcores=16)


def _hist_body(paths_hbm, zeros_hbm, ones_hbm, out_hbm, idxp, idxh, ones_v,
               zb, stage, cnt_sh, sem):
    core = lax.axis_index("c")
    sub = lax.axis_index("s")
    wid = core * 16 + sub
    pltpu.sync_copy(zeros_hbm, zb)
    pltpu.sync_copy(ones_hbm, ones_v)
    base = wid * EPW
    for hpass in range(2):
        lo = hpass * HCH
        for off, sz in ((0, 640), (640, 640), (1280, 328)):
            pltpu.sync_copy(zb.at[pl.ds(0, sz)],
                            cnt_sh.at[pl.ds(sub * 1608 + off, sz)])
        plsc.subcore_barrier()

        @pl.loop(0, STRIPS)
        def _strip(k):
            s = base + k * 128
            pltpu.sync_copy(paths_hbm.at[pl.ds(s, 128)], idxp)
            for g in range(8):
                pv = idxp[pl.ds(g * 16, 16)]
                inr = (pv >= lo) & (pv < lo + HCH)
                idxh[pl.ds(g * 16, 16)] = jnp.where(inr, pv - lo, HTRASH)
            pltpu.sync_copy(ones_v, cnt_sh.at[idxh], add=True)

        plsc.subcore_barrier()
        pltpu.sync_copy(cnt_sh.at[pl.ds(sub * 1600, 1600)], stage)
        pltpu.sync_copy(stage, out_hbm.at[core, pl.ds(lo + sub * 1600, 1600)])
        plsc.subcore_barrier()


@jax.jit
def _sc_hist(paths_p, zeros16, ones16):
    k = pl.kernel(
        _hist_body,
        out_type=jax.ShapeDtypeStruct((2, NPH, 16), jnp.float32),
        mesh=_get_mesh(),
        compiler_params=_SC_PARAMS,
        scratch_types=[
            pltpu.VMEM((128,), jnp.int32),
            pltpu.VMEM((128,), jnp.int32),
            pltpu.VMEM((128, 16), jnp.float32),
            pltpu.VMEM((640, 16), jnp.float32),
            pltpu.VMEM((1600, 16), jnp.float32),
            pltpu.VMEM_SHARED((HROWS, 16), jnp.float32),
            pltpu.SemaphoreType.DMA,
        ],
    )
    return k(paths_p, zeros16, ones16)


def _build_body(lsrc_hbm, links_hbm, paths_hbm, seq_hbm, bnd_hbm, zeros_hbm,
                xs_hbm, idxp, idxs, idxl, idxd, rows_v, zb, bnd_v, stage,
                buf_sh, sem):
    core = lax.axis_index("c")
    sub = lax.axis_index("s")
    is0 = core == 0
    pltpu.sync_copy(zeros_hbm, zb)
    pltpu.sync_copy(bnd_hbm, bnd_v)

    for li in range(NCH0):
        cbase = jnp.where(is0, li * CP, (NCH0 + li) * CP)
        lo0 = bnd_v[li, pl.ds(0, 16)][0]
        hi0 = bnd_v[li + 1, pl.ds(0, 16)][0]
        lo1 = bnd_v[NCH0 + li, pl.ds(0, 16)][0]
        hi1 = bnd_v[NCH0 + li + 1, pl.ds(0, 16)][0]
        lo = jnp.where(is0, lo0, lo1)
        hi = jnp.where(is0, hi0, hi1)
        for off, sz in ((0, 640), (640, 360)):
            pltpu.sync_copy(zb.at[pl.ds(0, sz)],
                            buf_sh.at[pl.ds(sub * CP + off, sz)])
        plsc.subcore_barrier()
        nE = hi - lo
        q = (nE + 15) // 16
        p0 = lo + jnp.minimum(sub * q, nE)
        p1 = lo + jnp.minimum(sub * q + q, nE)
        a = (p0 // 8) * 8
        nstrips = (p1 - a + 127) // 128

        def strip_body(k, _, a=a, p0=p0, p1=p1, cbase=cbase):
            s = a + k * 128
            pltpu.sync_copy(paths_hbm.at[pl.ds(s, 128)], idxp)
            pltpu.sync_copy(seq_hbm.at[pl.ds(s, 128)], idxs)
            pltpu.sync_copy(links_hbm.at[pl.ds(s, 128)], idxl)
            for g in range(8):
                ii = s + g * 16 + lax.iota(jnp.int32, 16)
                pv = idxp[pl.ds(g * 16, 16)]
                sv = idxs[pl.ds(g * 16, 16)]
                inr = (ii >= p0) & (ii < p1)
                inr &= (pv >= cbase) & (pv < cbase + CP)
                d = sv * CP + (pv - cbase)
                idxd[pl.ds(g * 16, 16)] = jnp.where(inr, d, TRASH)
            pltpu.async_copy(lsrc_hbm.at[idxl], rows_v, sem).wait()
            pltpu.sync_copy(rows_v, buf_sh.at[idxd], add=True)
            return 0

        lax.fori_loop(0, nstrips, strip_body, 0)
        plsc.subcore_barrier()
        pltpu.sync_copy(buf_sh.at[pl.ds(sub * CP, CP)], stage)
        pltpu.sync_copy(stage, xs_hbm.at[pl.ds(sub * NP + cbase, CP)])
        plsc.subcore_barrier()


@jax.jit
def _sc_build(lsrc, links_p, paths_p, seq_p, bnd, zeros640):
    k = pl.kernel(
        _build_body,
        out_type=jax.ShapeDtypeStruct((MS * NP, 32), jnp.float32),
        mesh=_get_mesh(),
        compiler_params=_SC_PARAMS,
        scratch_types=[
            pltpu.VMEM((128,), jnp.int32),
            pltpu.VMEM((128,), jnp.int32),
            pltpu.VMEM((128,), jnp.int32),
            pltpu.VMEM((128,), jnp.int32),
            pltpu.VMEM((128, 32), jnp.float32),
            pltpu.VMEM((640, 32), jnp.float32),
            pltpu.VMEM((64, 128), jnp.int32),
            pltpu.VMEM((CP, 32), jnp.float32),
            pltpu.VMEM_SHARED((CROWS + 8, 32), jnp.float32),
            pltpu.SemaphoreType.DMA,
        ],
    )
    return k(lsrc, links_p, paths_p, seq_p, bnd, zeros640)


def _agg_body(outs_hbm, links_hbm, paths_hbm, seq_hbm, zeros_hbm, out_hbm,
              idxp, idxs, idxl, idxf, rows_v, zb, stage, agg_sh, sem):
    core = lax.axis_index("c")
    sub = lax.axis_index("s")
    wid = core * 16 + sub
    pltpu.sync_copy(zeros_hbm, zb)
    pltpu.sync_copy(zb.at[pl.ds(0, AR)], agg_sh.at[pl.ds(sub * AR, AR)])
    plsc.subcore_barrier()
    base = wid * EPW

    @pl.loop(0, STRIPS)
    def _strip(k):
        s = base + k * 128
        pltpu.sync_copy(paths_hbm.at[pl.ds(s, 128)], idxp)
        pltpu.sync_copy(seq_hbm.at[pl.ds(s, 128)], idxs)
        pltpu.sync_copy(links_hbm.at[pl.ds(s, 128)], idxl)
        for g in range(8):
            pv = idxp[pl.ds(g * 16, 16)]
            sv = idxs[pl.ds(g * 16, 16)]
            idxf[pl.ds(g * 16, 16)] = sv * NP + pv
        pltpu.async_copy(outs_hbm.at[idxf], rows_v, sem).wait()
        pltpu.sync_copy(rows_v, agg_sh.at[idxl], add=True)

    plsc.subcore_barrier()
    pltpu.sync_copy(agg_sh.at[pl.ds(sub * AR, AR)], stage)
    pltpu.sync_copy(stage, out_hbm.at[core, pl.ds(sub * AR, AR)])


@jax.jit
def _sc_agg(outs_flat, links_p, paths_p, seq_p, zeros640):
    k = pl.kernel(
        _agg_body,
        out_type=jax.ShapeDtypeStruct((2, NLP, 32), jnp.float32),
        mesh=_get_mesh(),
        compiler_params=_SC_PARAMS,
        scratch_types=[
            pltpu.VMEM((128,), jnp.int32),
            pltpu.VMEM((128,), jnp.int32),
            pltpu.VMEM((128,), jnp.int32),
            pltpu.VMEM((128,), jnp.int32),
            pltpu.VMEM((128, 32), jnp.float32),
            pltpu.VMEM((640, 32), jnp.float32),
            pltpu.VMEM((AR, 32), jnp.float32),
            pltpu.VMEM_SHARED((NLP, 32), jnp.float32),
            pltpu.SemaphoreType.DMA,
        ],
    )
    return k(outs_flat, links_p, paths_p, seq_p, zeros640)


def _sigmoid(x):
    return 1.0 / (1.0 + jnp.exp(-x))


_SELU_SCALE = 1.0507009873554805
_SELU_ALPHA = 1.6732632423543772


def _selu(x):
    return _SELU_SCALE * jnp.where(x > 0, x, _SELU_ALPHA * (jnp.exp(x) - 1.0))


def _bnd_kernel(cnt_ref, seq_ref, bnd_ref):
    s = cnt_ref[0] + cnt_ref[1]
    hp = lax.Precision.HIGHEST
    rowsum = jnp.dot(s, jnp.ones((128, 1), jnp.float32),
                     precision=hp)
    vr = lax.broadcasted_iota(jnp.int32, (NCH, NP // 8), 1)
    cc = lax.broadcasted_iota(jnp.int32, (NCH, NP // 8), 0)
    sel = (vr // (CP // 8) == cc).astype(jnp.float32)
    cs = jnp.dot(sel, rowsum[:NP // 8], precision=hp) * (1.0 / 16.0)
    csb = jnp.broadcast_to(cs, (NCH, 128))
    ii = lax.broadcasted_iota(jnp.int32, (NCH + 1, NCH), 0)
    jj = lax.broadcasted_iota(jnp.int32, (NCH + 1, NCH), 1)
    tri = (jj < ii).astype(jnp.float32)
    bnd = jnp.dot(tri, csb, precision=hp)
    maxlen = (jnp.max(seq_ref[...]) + 1).astype(jnp.float32)
    mrow = jnp.full((1, 128), maxlen, jnp.float32)
    out = jnp.concatenate(
        [bnd, mrow, jnp.zeros((64 - NCH - 2, 128), jnp.float32)], axis=0)
    bnd_ref[...] = out.astype(jnp.int32)


@jax.jit
def _tc_bnd(cnt2, seq2d):
    return pl.pallas_call(
        _bnd_kernel,
        in_specs=[
            pl.BlockSpec((2, NPH // 8, 128), lambda: (0, 0, 0)),
            pl.BlockSpec(seq2d.shape, lambda: (0, 0)),
        ],
        out_specs=pl.BlockSpec((64, 128), lambda: (0, 0)),
        out_shape=jax.ShapeDtypeStruct((64, 128), jnp.int32),
    )(cnt2, seq2d)


_BP = 2000


def _mask_kernel(cnt_ref, bnd_ref, mask_ref):
    lens = cnt_ref[0] + cnt_ref[1]
    lens_col = lens[:, 0:1]
    maxlen = bnd_ref[NCH + 1:NCH + 2, 0:1].astype(jnp.float32)
    leff = jnp.minimum(lens_col, maxlen).astype(jnp.int32)
    tt = lax.broadcasted_iota(jnp.int32, (_BP, MS), 1)
    mask_ref[...] = (tt < leff).astype(jnp.float32)


@jax.jit
def _tc_mask(counts, bnd):
    return pl.pallas_call(
        _mask_kernel,
        grid=(NP // _BP,),
        in_specs=[
            pl.BlockSpec((2, _BP, 16), lambda i: (0, i, 0)),
            pl.BlockSpec((64, 128), lambda i: (0, 0)),
        ],
        out_specs=pl.BlockSpec((_BP, MS), lambda i: (i, 0)),
        out_shape=jax.ShapeDtypeStruct((NP, MS), jnp.float32),
    )(counts, bnd)


_GB = 1000


def _gru_scan_kernel(xs_ref, mask_ref, h_ref, wx_ref, wh_ref, bx_ref, bh_ref,
                     outs_ref, hout_ref):
    h = h_ref[...]
    wx = wx_ref[...]
    wh = wh_ref[...]
    bx = bx_ref[...]
    bh = bh_ref[...]
    mblk = mask_ref[...]
    for t in range(MS):
        xt = xs_ref[t] @ wx + bx
        hs = h @ wh + bh
        z = _sigmoid(xt[:, :P] + hs[:, :P])
        r = _sigmoid(xt[:, P:2 * P] + hs[:, P:2 * P])
        hh = jnp.tanh(xt[:, 2 * P:] + r * hs[:, 2 * P:])
        hn = z * h + (1.0 - z) * hh
        m = mblk[:, t:t + 1]
        h = m * hn + (1.0 - m) * h
        outs_ref[t] = h
    hout_ref[...] = h


@jax.jit
def _tc_gru(xs3, mask, h, wx, wh, bx, bh):
    grid = (NP // _GB,)
    return pl.pallas_call(
        _gru_scan_kernel,
        grid=grid,
        in_specs=[
            pl.BlockSpec((MS, _GB, 32), lambda i: (0, i, 0)),
            pl.BlockSpec((_GB, MS), lambda i: (i, 0)),
            pl.BlockSpec((_GB, P), lambda i: (i, 0)),
            pl.BlockSpec((P, 3 * P), lambda i: (0, 0)),
            pl.BlockSpec((P, 3 * P), lambda i: (0, 0)),
            pl.BlockSpec((1, 3 * P), lambda i: (0, 0)),
            pl.BlockSpec((1, 3 * P), lambda i: (0, 0)),
        ],
        out_specs=[
            pl.BlockSpec((MS, _GB, 32), lambda i: (0, i, 0)),
            pl.BlockSpec((_GB, P), lambda i: (i, 0)),
        ],
        out_shape=[
            jax.ShapeDtypeStruct((MS, NP, 32), jnp.float32),
            jax.ShapeDtypeStruct((NP, P), jnp.float32),
        ],
    )(xs3, mask, h, wx, wh, bx, bh)


def _link_kernel(agg_ref, h_ref, wx_ref, wh_ref, bx_ref, bh_ref, o_ref):
    x = agg_ref[0] + agg_ref[1]
    h = h_ref[...]
    xs = x @ wx_ref[...] + bx_ref[...]
    hs = h @ wh_ref[...] + bh_ref[...]
    z = _sigmoid(xs[:, :L] + hs[:, :L])
    r = _sigmoid(xs[:, L:2 * L] + hs[:, L:2 * L])
    hh = jnp.tanh(xs[:, 2 * L:] + r * hs[:, 2 * L:])
    o_ref[...] = z * h + (1.0 - z) * hh


@jax.jit
def _tc_link(agg2, lstate, wx, wh, bx, bh):
    return pl.pallas_call(
        _link_kernel,
        in_specs=[
            pl.BlockSpec((2, NLP, 32), lambda: (0, 0, 0)),
            pl.BlockSpec((NLP, 32), lambda: (0, 0)),
            pl.BlockSpec((P, 3 * L), lambda: (0, 0)),
            pl.BlockSpec((L, 3 * L), lambda: (0, 0)),
            pl.BlockSpec((1, 3 * L), lambda: (0, 0)),
            pl.BlockSpec((1, 3 * L), lambda: (0, 0)),
        ],
        out_specs=pl.BlockSpec((NLP, 32), lambda: (0, 0)),
        out_shape=jax.ShapeDtypeStruct((NLP, 32), jnp.float32),
    )(agg2, lstate, wx, wh, bx, bh)


def _mlp_kernel(x_ref, w1_ref, b1_ref, w2_ref, b2_ref, w3_ref, b3_ref, o_ref):
    x = x_ref[...]
    r = x @ w1_ref[...] + b1_ref[...]
    r = _selu(r)
    r = r @ w2_ref[...] + b2_ref[...]
    r = jnp.maximum(r, 0.0)
    o_ref[...] = r @ w3_ref[...] + b3_ref[...]


_RB = 2000


@jax.jit
def _readout(path_state, W1, b1, W2, b2, W3, b3):
    return pl.pallas_call(
        _mlp_kernel,
        grid=(NP // _RB,),
        in_specs=[
            pl.BlockSpec((_RB, P), lambda i: (i, 0)),
            pl.BlockSpec((P, RU), lambda i: (0, 0)),
            pl.BlockSpec((1, RU), lambda i: (0, 0)),
            pl.BlockSpec((RU, RU), lambda i: (0, 0)),
            pl.BlockSpec((1, RU), lambda i: (0, 0)),
            pl.BlockSpec((RU, 1), lambda i: (0, 0)),
            pl.BlockSpec((1, 1), lambda i: (0, 0)),
        ],
        out_specs=pl.BlockSpec((_RB, 1), lambda i: (i, 0)),
        out_shape=jax.ShapeDtypeStruct((NP, 1), jnp.float32),
    )(path_state, W1, b1[None, :], W2, b2[None, :], W3, b3[None, :])


def kernel(links, paths, sequences, link_capacity, bandwith,
           path_Wx, path_Wh, path_bx, path_bh,
           link_Wx, link_Wh, link_bx, link_bh,
           W1, b1, W2, b2, W3, b3):
    f32 = jnp.float32
    i32 = jnp.int32
    npad = EPAD - E
    links_p = jnp.concatenate([links.astype(i32), jnp.full((npad,), NL, i32)])
    paths_p = jnp.concatenate([paths.astype(i32), jnp.full((npad,), NP, i32)])
    seq_p = jnp.concatenate([sequences.astype(i32), jnp.zeros((npad,), i32)])
    zeros640 = jnp.zeros((640, 32), f32)
    zeros16 = jnp.zeros((640, 16), f32)
    ones16 = jnp.ones((128, 16), f32)

    lstate = jnp.concatenate(
        [link_capacity[:, None], jnp.zeros((NL, L - 1), f32)], axis=1)
    lstate = jnp.concatenate([lstate, jnp.zeros((NLP - NL, L), f32)], axis=0)
    pstate = jnp.concatenate(
        [bandwith[:, None], jnp.zeros((NP, P - 1), f32)], axis=1)

    counts = _sc_hist(paths_p, zeros16, ones16)
    bnd = _tc_bnd(counts.reshape(2, NPH // 8, 128), seq_p.reshape(EPAD // 128, 128))
    mask = _tc_mask(counts, bnd)

    pbx = path_bx[None, :]
    pbh = path_bh[None, :]
    lbx = link_bx[None, :]
    lbh = link_bh[None, :]

    for _ in range(T):
        xs = _sc_build(lstate, links_p, paths_p, seq_p, bnd, zeros640)
        outs, pstate = _tc_gru(xs.reshape(MS, NP, 32), mask, pstate,
                               path_Wx, path_Wh, pbx, pbh)
        agg2 = _sc_agg(outs.reshape(MS * NP, 32), links_p, paths_p, seq_p,
                       zeros640)
        lstate = _tc_link(agg2, lstate, link_Wx, link_Wh, lbx, lbh)

    return _readout(pstate, W1, b1, W2, b2, W3, b3)
